# speedup vs baseline: 56.8565x; 56.8565x over previous
import functools

import jax
import jax.numpy as jnp
from jax import lax
from jax.experimental import pallas as pl
from jax.experimental.pallas import tpu as pltpu
from jax.experimental.pallas import tpu_sc as plsc

VOCAB_PAD = 1024
CH = 2048
GCH = 128
LANES = 16


def _table_body(scale_ref, w_ref, out_ref):
    w = w_ref[...]
    nsq = jnp.sum(w * w, axis=1, keepdims=True)
    inv = 1.0 / jnp.maximum(jnp.sqrt(nsq), 1e-8)
    nrm = w * inv
    g = lax.dot_general(nrm, nrm, (((1,), (1,)), ((), ())),
                        preferred_element_type=jnp.float32,
                        precision=lax.Precision.HIGHEST)
    out_ref[...] = g * scale_ref[0]


def _build_table(w_pad, scale):
    return pl.pallas_call(
        _table_body,
        out_shape=jax.ShapeDtypeStruct((VOCAB_PAD, VOCAB_PAD), jnp.float32),
        in_specs=[
            pl.BlockSpec(memory_space=pltpu.SMEM),
            pl.BlockSpec(memory_space=pltpu.VMEM),
        ],
        out_specs=pl.BlockSpec(memory_space=pltpu.VMEM),
    )(scale, w_pad)


@functools.lru_cache(maxsize=None)
def _make_gather(num_pos, num_neg, neg_rate):
    info = plsc.get_sparse_core_info()
    nc, ns = info.num_cores, info.num_subcores
    nw = nc * ns
    p_cnt = num_pos // nw
    n_cnt = num_neg // nw
    assert p_cnt * nw == num_pos and n_cnt * nw == num_neg
    assert p_cnt % CH == 0 and n_cnt % CH == 0
    mesh = plsc.VectorSubcoreMesh(core_axis_name="c", subcore_axis_name="s")

    @functools.partial(
        pl.kernel, mesh=mesh,
        out_type=jax.ShapeDtypeStruct((nw, LANES), jnp.float32),
        scratch_types=[
            pltpu.VMEM((CH,), jnp.int32),
            pltpu.VMEM((CH,), jnp.int32),
            pltpu.VMEM((CH,), jnp.int32),
            pltpu.VMEM((CH,), jnp.float32),
            pltpu.VMEM((LANES,), jnp.float32),
            pltpu.SemaphoreType.DMA,
            pltpu.SemaphoreType.DMA,
        ],
    )
    def gather_kernel(table_hbm, cen_p_hbm, pos_hbm, cen_n_hbm, neg_hbm,
                      out_hbm, cen_buf, oth_buf, idx_buf, val_buf, acc_buf,
                      sem_in, sem_g):
        wid = lax.axis_index("s") * nc + lax.axis_index("c")

        def accum_range(cen_hbm, oth_hbm, start, nchunks):
            def chunk_body(ci, acc):
                off = start + ci * CH
                cp1 = pltpu.async_copy(cen_hbm.at[pl.ds(off, CH)], cen_buf,
                                       sem_in)
                cp2 = pltpu.async_copy(oth_hbm.at[pl.ds(off, CH)], oth_buf,
                                       sem_in)
                cp1.wait()
                cp2.wait()

                def idx_body(vi, carry):
                    st = vi * LANES
                    idx_buf[pl.ds(st, LANES)] = (
                        cen_buf[pl.ds(st, LANES)] * VOCAB_PAD
                        + oth_buf[pl.ds(st, LANES)])
                    return carry

                lax.fori_loop(0, CH // LANES, idx_body, 0)

                copies = [
                    pltpu.async_copy(
                        table_hbm.at[idx_buf.at[pl.ds(j * GCH, GCH)]],
                        val_buf.at[pl.ds(j * GCH, GCH)], sem_g)
                    for j in range(CH // GCH)
                ]
                for cp in copies:
                    cp.wait()

                def acc_body(vi, a):
                    return a + val_buf[pl.ds(vi * LANES, LANES)]

                return lax.fori_loop(0, CH // LANES, acc_body, acc)

            return lax.fori_loop(0, nchunks, chunk_body,
                                 jnp.zeros((LANES,), jnp.float32))

        acc_p = accum_range(cen_p_hbm, pos_hbm, wid * p_cnt, p_cnt // CH)
        acc_n = accum_range(cen_n_hbm, neg_hbm, wid * n_cnt, n_cnt // CH)
        acc_buf[...] = acc_n - float(neg_rate) * acc_p
        pltpu.sync_copy(acc_buf, out_hbm.at[wid])

    return gather_kernel


def kernel(center, pos_word, neg_word, w2v):
    b, s = center.shape
    r = neg_word.shape[1] // s
    v = w2v.shape[0]
    scale = jnp.full((1,), 1.0 / (s * r * b * 2.0), dtype=jnp.float32)
    w_pad = jnp.pad(w2v, ((0, VOCAB_PAD - v), (0, 0)))
    table = _build_table(w_pad, scale).reshape(-1)
    cen_p = center.reshape(-1)
    pos_f = pos_word.reshape(-1)
    cen_n = jnp.tile(center, (1, r)).reshape(-1)
    neg_f = neg_word.reshape(-1)
    gather = _make_gather(cen_p.size, neg_f.size, r)
    partials = gather(table, cen_p, pos_f, cen_n, neg_f)
    return 1.0 + jnp.sum(partials)


# speedup vs baseline: 56.8587x; 1.0000x over previous
import functools

import jax
import jax.numpy as jnp
from jax import lax
from jax.experimental import pallas as pl
from jax.experimental.pallas import tpu as pltpu
from jax.experimental.pallas import tpu_sc as plsc

VOCAB_PAD = 1024
CH = 2048
GCH = 128
LANES = 16


def _table_body(scale_ref, w_ref, out_ref):
    w = w_ref[...]
    nsq = jnp.sum(w * w, axis=1, keepdims=True)
    inv = 1.0 / jnp.maximum(jnp.sqrt(nsq), 1e-8)
    nrm = w * inv
    g = lax.dot_general(nrm, nrm, (((1,), (1,)), ((), ())),
                        preferred_element_type=jnp.float32,
                        precision=lax.Precision.HIGHEST)
    out_ref[...] = g * scale_ref[0]


def _build_table(w_pad, scale):
    return pl.pallas_call(
        _table_body,
        out_shape=jax.ShapeDtypeStruct((VOCAB_PAD, VOCAB_PAD), jnp.float32),
        in_specs=[
            pl.BlockSpec(memory_space=pltpu.SMEM),
            pl.BlockSpec(memory_space=pltpu.VMEM),
        ],
        out_specs=pl.BlockSpec(memory_space=pltpu.VMEM),
    )(scale, w_pad)


@functools.lru_cache(maxsize=None)
def _make_gather(num_pos, num_neg, neg_rate):
    info = plsc.get_sparse_core_info()
    nc, ns = info.num_cores, info.num_subcores
    nw = nc * ns
    p_cnt = num_pos // nw
    n_cnt = num_neg // nw
    assert p_cnt * nw == num_pos and n_cnt * nw == num_neg
    assert p_cnt % CH == 0 and n_cnt % CH == 0
    mesh = plsc.VectorSubcoreMesh(core_axis_name="c", subcore_axis_name="s")

    @functools.partial(
        pl.kernel, mesh=mesh,
        out_type=jax.ShapeDtypeStruct((nw, LANES), jnp.float32),
        scratch_types=[
            pltpu.VMEM((CH,), jnp.int32),
            pltpu.VMEM((CH,), jnp.int32),
            pltpu.VMEM((CH,), jnp.int32),
            pltpu.VMEM((CH,), jnp.float32),
            pltpu.VMEM((LANES,), jnp.float32),
            pltpu.SemaphoreType.DMA,
            pltpu.SemaphoreType.DMA,
        ],
    )
    def gather_kernel(table_hbm, cen_p_hbm, pos_hbm, cen_n_hbm, neg_hbm,
                      out_hbm, cen_buf, oth_buf, idx_buf, val_buf, acc_buf,
                      sem_in, sem_g):
        wid = lax.axis_index("s") * nc + lax.axis_index("c")

        def accum_range(cen_hbm, oth_hbm, start, nchunks):
            def chunk_body(ci, acc):
                off = start + ci * CH
                cp1 = pltpu.async_copy(cen_hbm.at[pl.ds(off, CH)], cen_buf,
                                       sem_in)
                cp2 = pltpu.async_copy(oth_hbm.at[pl.ds(off, CH)], oth_buf,
                                       sem_in)
                cp1.wait()
                cp2.wait()

                def idx_body(vi, carry):
                    st = vi * LANES
                    idx_buf[pl.ds(st, LANES)] = (
                        cen_buf[pl.ds(st, LANES)] * VOCAB_PAD
                        + oth_buf[pl.ds(st, LANES)])
                    return carry

                lax.fori_loop(0, CH // LANES, idx_body, 0)

                copies = [
                    pltpu.async_copy(
                        table_hbm.at[idx_buf.at[pl.ds(j * GCH, GCH)]],
                        val_buf.at[pl.ds(j * GCH, GCH)], sem_g)
                    for j in range(CH // GCH)
                ]
                for cp in copies:
                    cp.wait()

                def acc_body(vi, a):
                    return a + val_buf[pl.ds(vi * LANES, LANES)]

                return lax.fori_loop(0, CH // LANES, acc_body, acc)

            return lax.fori_loop(0, nchunks, chunk_body,
                                 jnp.zeros((LANES,), jnp.float32))

        acc_p = accum_range(cen_p_hbm, pos_hbm, wid * p_cnt, p_cnt // CH)
        acc_n = accum_range(cen_n_hbm, neg_hbm, wid * n_cnt, n_cnt // CH)
        acc_buf[...] = acc_n - float(neg_rate) * acc_p
        pltpu.sync_copy(acc_buf, out_hbm.at[wid])

    return gather_kernel


def kernel(center, pos_word, neg_word, w2v):
    b, s = center.shape
    r = neg_word.shape[1] // s
    v = w2v.shape[0]
    scale = jnp.full((1,), 1.0 / (s * r * b * 2.0), dtype=jnp.float32)
    w_pad = jnp.pad(w2v, ((0, VOCAB_PAD - v), (0, 0)))
    table = _build_table(w_pad, scale).reshape(-1)
    cen_p = center.reshape(-1)
    pos_f = pos_word.reshape(-1)
    cen_n = jnp.broadcast_to(center[:, None, :], (b, r, s)).reshape(-1)
    neg_f = neg_word.reshape(-1)
    gather = _make_gather(cen_p.size, neg_f.size, r)
    partials = gather(table, cen_p, pos_f, cen_n, neg_f)
    return 1.0 + jnp.sum(partials)


# speedup vs baseline: 62.7365x; 1.1034x over previous
import functools

import jax
import jax.numpy as jnp
from jax import lax
from jax.experimental import pallas as pl
from jax.experimental.pallas import tpu as pltpu
from jax.experimental.pallas import tpu_sc as plsc

VOCAB_PAD = 1024
CH = 2048
GCH = 128
LANES = 16


def _table_body(scale_ref, w_ref, out_ref):
    w = w_ref[...]
    nsq = jnp.sum(w * w, axis=1, keepdims=True)
    inv = 1.0 / jnp.maximum(jnp.sqrt(nsq), 1e-8)
    nrm = w * inv
    g = lax.dot_general(nrm, nrm, (((1,), (1,)), ((), ())),
                        preferred_element_type=jnp.float32,
                        precision=lax.Precision.HIGHEST)
    out_ref[...] = g * scale_ref[0]


def _build_table(w_pad, scale):
    return pl.pallas_call(
        _table_body,
        out_shape=jax.ShapeDtypeStruct((VOCAB_PAD, VOCAB_PAD), jnp.float32),
        in_specs=[
            pl.BlockSpec(memory_space=pltpu.SMEM),
            pl.BlockSpec(memory_space=pltpu.VMEM),
        ],
        out_specs=pl.BlockSpec(memory_space=pltpu.VMEM),
    )(scale, w_pad)


@functools.lru_cache(maxsize=None)
def _make_gather(num_pos, num_neg, neg_rate):
    info = plsc.get_sparse_core_info()
    nc, ns = info.num_cores, info.num_subcores
    nw = nc * ns
    p_cnt = num_pos // nw
    n_cnt = num_neg // nw
    assert p_cnt * nw == num_pos and n_cnt * nw == num_neg
    assert p_cnt % CH == 0 and n_cnt % CH == 0
    mesh = plsc.VectorSubcoreMesh(core_axis_name="c", subcore_axis_name="s")

    @functools.partial(
        pl.kernel, mesh=mesh,
        out_type=jax.ShapeDtypeStruct((nw, LANES), jnp.float32),
        scratch_types=[
            pltpu.VMEM((p_cnt,), jnp.int32),
            pltpu.VMEM((CH,), jnp.int32),
            pltpu.VMEM((CH,), jnp.int32),
            pltpu.VMEM((CH,), jnp.float32),
            pltpu.VMEM((LANES,), jnp.float32),
            pltpu.SemaphoreType.DMA,
            pltpu.SemaphoreType.DMA,
        ],
    )
    def gather_kernel(table_hbm, cen_hbm, pos_hbm, negt_hbm,
                      out_hbm, cen_buf, oth_buf, idx_buf, val_buf, acc_buf,
                      sem_in, sem_g):
        wid = lax.axis_index("s") * nc + lax.axis_index("c")
        pltpu.async_copy(cen_hbm.at[pl.ds(wid * p_cnt, p_cnt)], cen_buf,
                         sem_in).wait()

        def accum_range(oth_hbm, start, acc):
            def chunk_body(ci, acc):
                cp2 = pltpu.async_copy(
                    oth_hbm.at[pl.ds(start + ci * CH, CH)], oth_buf, sem_in)
                cp2.wait()

                def idx_body(vi, carry):
                    st = vi * LANES
                    idx_buf[pl.ds(st, LANES)] = (
                        cen_buf[pl.ds(ci * CH + st, LANES)] * VOCAB_PAD
                        + oth_buf[pl.ds(st, LANES)])
                    return carry

                lax.fori_loop(0, CH // LANES, idx_body, 0)

                copies = [
                    pltpu.async_copy(
                        table_hbm.at[idx_buf.at[pl.ds(j * GCH, GCH)]],
                        val_buf.at[pl.ds(j * GCH, GCH)], sem_g)
                    for j in range(CH // GCH)
                ]
                for cp in copies:
                    cp.wait()

                def acc_body(vi, a):
                    return a + val_buf[pl.ds(vi * LANES, LANES)]

                return lax.fori_loop(0, CH // LANES, acc_body, acc)

            return lax.fori_loop(0, p_cnt // CH, chunk_body, acc)

        acc_p = accum_range(pos_hbm, wid * p_cnt,
                            jnp.zeros((LANES,), jnp.float32))
        acc_n = jnp.zeros((LANES,), jnp.float32)
        for rr in range(neg_rate):
            acc_n = accum_range(negt_hbm, rr * num_pos + wid * p_cnt, acc_n)
        acc_buf[...] = acc_n - float(neg_rate) * acc_p
        pltpu.sync_copy(acc_buf, out_hbm.at[wid])

    return gather_kernel


def kernel(center, pos_word, neg_word, w2v):
    b, s = center.shape
    r = neg_word.shape[1] // s
    v = w2v.shape[0]
    scale = jnp.full((1,), 1.0 / (s * r * b * 2.0), dtype=jnp.float32)
    w_pad = jnp.pad(w2v, ((0, VOCAB_PAD - v), (0, 0)))
    table = _build_table(w_pad, scale).reshape(-1)
    cen_f = center.reshape(-1)
    pos_f = pos_word.reshape(-1)
    negt_f = jnp.swapaxes(neg_word.reshape(b, r, s), 0, 1).reshape(-1)
    gather = _make_gather(cen_f.size, negt_f.size, r)
    partials = gather(table, cen_f, pos_f, negt_f)
    return 1.0 + jnp.sum(partials)


# speedup vs baseline: 84.1925x; 1.3420x over previous
import functools

import jax
import jax.numpy as jnp
from jax import lax
from jax.experimental import pallas as pl
from jax.experimental.pallas import tpu as pltpu
from jax.experimental.pallas import tpu_sc as plsc

VOCAB_PAD = 1024
GCH = 128
LANES = 16


def _table_body(scale_ref, w_ref, out_ref):
    w = w_ref[...]
    nsq = jnp.sum(w * w, axis=1, keepdims=True)
    inv = 1.0 / jnp.maximum(jnp.sqrt(nsq), 1e-8)
    nrm = w * inv
    g = lax.dot_general(nrm, nrm, (((1,), (1,)), ((), ())),
                        preferred_element_type=jnp.float32,
                        precision=lax.Precision.HIGHEST)
    out_ref[...] = g * scale_ref[0]


def _build_table(w_pad, scale):
    return pl.pallas_call(
        _table_body,
        out_shape=jax.ShapeDtypeStruct((VOCAB_PAD, VOCAB_PAD), jnp.float32),
        in_specs=[
            pl.BlockSpec(memory_space=pltpu.SMEM),
            pl.BlockSpec(memory_space=pltpu.VMEM),
        ],
        out_specs=pl.BlockSpec(memory_space=pltpu.VMEM),
    )(scale, w_pad)


@functools.lru_cache(maxsize=None)
def _make_gather(num_pos, num_neg, neg_rate):
    info = plsc.get_sparse_core_info()
    nc, ns = info.num_cores, info.num_subcores
    nw = nc * ns
    p_cnt = num_pos // nw
    assert p_cnt * nw == num_pos and num_neg == num_pos * neg_rate
    assert p_cnt % GCH == 0 and p_cnt % LANES == 0
    nph = 1 + neg_rate
    mesh = plsc.VectorSubcoreMesh(core_axis_name="c", subcore_axis_name="s")

    @functools.partial(
        pl.kernel, mesh=mesh,
        out_type=jax.ShapeDtypeStruct((nw, LANES), jnp.float32),
        scratch_types=[
            pltpu.VMEM((p_cnt,), jnp.int32),
            pltpu.VMEM((p_cnt,), jnp.int32),
            pltpu.VMEM((neg_rate * p_cnt,), jnp.int32),
            pltpu.VMEM((p_cnt,), jnp.int32),
            pltpu.VMEM((p_cnt,), jnp.int32),
            pltpu.VMEM((p_cnt,), jnp.float32),
            pltpu.VMEM((p_cnt,), jnp.float32),
            pltpu.VMEM((LANES,), jnp.float32),
            pltpu.SemaphoreType.DMA,
            pltpu.SemaphoreType.DMA,
            pltpu.SemaphoreType.DMA,
            pltpu.SemaphoreType.DMA,
        ],
    )
    def gather_kernel(table_hbm, cen_hbm, pos_hbm, negt_hbm, out_hbm,
                      cb, pb, nb, ix_a, ix_b, vl_a, vl_b, accb,
                      sem_cp, sem_ng, sem_a, sem_b):
        wid = lax.axis_index("s") * nc + lax.axis_index("c")
        base = wid * p_cnt
        cp_c = pltpu.async_copy(cen_hbm.at[pl.ds(base, p_cnt)], cb, sem_cp)
        cp_p = pltpu.async_copy(pos_hbm.at[pl.ds(base, p_cnt)], pb, sem_cp)
        cp_n = [
            pltpu.async_copy(
                negt_hbm.at[pl.ds(rr * num_pos + base, p_cnt)],
                nb.at[pl.ds(rr * p_cnt, p_cnt)], sem_ng)
            for rr in range(neg_rate)
        ]
        cp_c.wait()
        cp_p.wait()

        def compute_idx(oth_ref, oth_off, ix_ref):
            def body(vi, carry):
                st = vi * LANES
                ix_ref[pl.ds(st, LANES)] = (
                    cb[pl.ds(st, LANES)] * VOCAB_PAD
                    + oth_ref[pl.ds(oth_off + st, LANES)])
                return carry
            lax.fori_loop(0, p_cnt // LANES, body, 0)

        def fire(ix_ref, vl_ref, sem):
            return [
                pltpu.async_copy(
                    table_hbm.at[ix_ref.at[pl.ds(j * GCH, GCH)]],
                    vl_ref.at[pl.ds(j * GCH, GCH)], sem)
                for j in range(p_cnt // GCH)
            ]

        def acc_into(vl_ref, acc):
            def body(vi, a):
                return a + vl_ref[pl.ds(vi * LANES, LANES)]
            return lax.fori_loop(0, p_cnt // LANES, body, acc)

        ix = [ix_a, ix_b]
        vl = [vl_a, vl_b]
        sems = [sem_a, sem_b]

        acc_p = jnp.zeros((LANES,), jnp.float32)
        acc_n = jnp.zeros((LANES,), jnp.float32)

        compute_idx(pb, 0, ix[0])
        inflight = fire(ix[0], vl[0], sems[0])
        for p in range(1, nph):
            if p == 1:
                for cp in cp_n:
                    cp.wait()
            compute_idx(nb, (p - 1) * p_cnt, ix[p % 2])
            nxt = fire(ix[p % 2], vl[p % 2], sems[p % 2])
            for cp in inflight:
                cp.wait()
            if p - 1 == 0:
                acc_p = acc_into(vl[0], acc_p)
            else:
                acc_n = acc_into(vl[(p - 1) % 2], acc_n)
            inflight = nxt
        for cp in inflight:
            cp.wait()
        acc_n = acc_into(vl[(nph - 1) % 2], acc_n)

        accb[...] = acc_n - float(neg_rate) * acc_p
        pltpu.sync_copy(accb, out_hbm.at[wid])

    return gather_kernel


def kernel(center, pos_word, neg_word, w2v):
    b, s = center.shape
    r = neg_word.shape[1] // s
    v = w2v.shape[0]
    scale = jnp.full((1,), 1.0 / (s * r * b * 2.0), dtype=jnp.float32)
    w_pad = jnp.pad(w2v, ((0, VOCAB_PAD - v), (0, 0)))
    table = _build_table(w_pad, scale).reshape(-1)
    cen_f = center.reshape(-1)
    pos_f = pos_word.reshape(-1)
    negt_f = jnp.swapaxes(neg_word.reshape(b, r, s), 0, 1).reshape(-1)
    gather = _make_gather(cen_f.size, negt_f.size, r)
    partials = gather(table, cen_f, pos_f, negt_f)
    return 1.0 + jnp.sum(partials)
